# Initial kernel scaffold; baseline (speedup 1.0000x reference)
import jax
import jax.numpy as jnp
from jax.experimental import pallas as pl


def kernel(x, edge_index, edge_attr, Wcz, bcz, Wcr, bcr, Wch, bch, Wlz, blz, Wlr, blr, Wlh, blh, attention, Wlin, blin):
    raise NotImplementedError("write your pallas kernel here")


# speedup vs baseline: 59.6481x; 59.6481x over previous
import functools

import jax
import jax.numpy as jnp
from jax import lax
from jax.experimental import pallas as pl
from jax.experimental.pallas import tpu as pltpu
from jax.experimental.pallas import tpu_sc as plsc

_N = 100000
_E = 3200000
_HID = 32
_PERIODS = 12

_NSUB = 16
_NCORE = 2
_NP = 100352
_NPH = _NP // 2
_ZR = _NPH // _NSUB
_CR = 16
_RPT = 1568
_CH = _RPT // _CR
_ROWS = _NSUB * _RPT
_EPAD = _ROWS * 128
_B = 2000


def _sc_mesh():
    return plsc.VectorSubcoreMesh(core_axis_name="c", subcore_axis_name="s")


def _remap_half(idx, lo):
    idx2 = idx - lo
    ok = jnp.logical_and(idx2 >= 0, idx2 < _NPH)
    return jnp.where(ok, idx2, _NPH)


@functools.partial(
    pl.kernel,
    out_type=pltpu.HBM((_NCORE, _NPH, 16), jnp.float32),
    mesh=_sc_mesh(),
    compiler_params=pltpu.CompilerParams(use_tc_tiling_on_sc=False),
    scratch_types=[
        pltpu.VMEM((_CR, 128), jnp.int32),
        pltpu.VMEM((_CR, 128), jnp.int32),
        pltpu.VMEM((_CR, 128), jnp.float32),
        pltpu.VMEM((128, 16), jnp.float32),
        pltpu.VMEM((_ZR, 16), jnp.float32),
        pltpu.VMEM_SHARED((_NPH + 8, 16), jnp.float32),
    ],
)
def _deg_kernel(col_hbm, w_hbm, out_hbm, cidx, cidx2, wbuf, rows, zbuf,
                shared):
    c = lax.axis_index("c")
    s = lax.axis_index("s")
    lo = c * _NPH

    zero16 = jnp.zeros((16,), jnp.float32)

    def zb(i, carry):
        zbuf[i, :] = zero16
        return carry

    lax.fori_loop(0, _ZR, zb, 0)
    pltpu.sync_copy(zbuf, shared.at[pl.ds(s * _ZR, _ZR)])
    plsc.subcore_barrier()

    def chunk(ch, carry):
        base = s * _RPT + ch * _CR
        pltpu.sync_copy(col_hbm.at[pl.ds(base, _CR)], cidx)
        pltpu.sync_copy(w_hbm.at[pl.ds(base, _CR)], wbuf)

        def gbody(g, gc):
            for j in range(8):
                iv = cidx[g, pl.ds(j * 16, 16)]
                cidx2[g, pl.ds(j * 16, 16)] = _remap_half(iv, lo)
                wv = wbuf[g, pl.ds(j * 16, 16)]
                for u in range(16):
                    rows[j * 16 + u, :] = jnp.full((16,), wv[u], jnp.float32)
            pltpu.sync_copy(rows, shared.at[cidx2.at[g]], add=True)
            return gc

        lax.fori_loop(0, _CR, gbody, 0)
        return carry

    lax.fori_loop(0, _CH, chunk, 0)
    plsc.subcore_barrier()
    pltpu.sync_copy(
        shared.at[pl.ds(s * _ZR, _ZR)], out_hbm.at[c, pl.ds(s * _ZR, _ZR)]
    )


@functools.partial(
    pl.kernel,
    out_type=pltpu.HBM((_NCORE, 3, _NPH, 16), jnp.float32),
    mesh=_sc_mesh(),
    compiler_params=pltpu.CompilerParams(use_tc_tiling_on_sc=False),
    scratch_types=[
        pltpu.VMEM((_CR, 128), jnp.int32),
        pltpu.VMEM((_CR, 128), jnp.int32),
        pltpu.VMEM((_CR, 128), jnp.int32),
        pltpu.VMEM((_CR, 128), jnp.float32),
        pltpu.VMEM((128, 16), jnp.float32),
        pltpu.VMEM((_ZR, 16), jnp.float32),
        pltpu.VMEM_SHARED((_NPH + 8, 16), jnp.float32),
    ],
)
def _agg_kernel(row_hbm, col_hbm, w_hbm, xs0, xs1, xs2, out_hbm,
                ridx, cidx, cidx2, wbuf, rows, zbuf, shared):
    c = lax.axis_index("c")
    s = lax.axis_index("s")
    lo = c * _NPH

    zero16 = jnp.zeros((16,), jnp.float32)

    def zb(i, carry):
        zbuf[i, :] = zero16
        return carry

    lax.fori_loop(0, _ZR, zb, 0)

    for s3, xs in enumerate((xs0, xs1, xs2)):
        pltpu.sync_copy(zbuf, shared.at[pl.ds(s * _ZR, _ZR)])
        plsc.subcore_barrier()

        def chunk(ch, carry):
            base = s * _RPT + ch * _CR
            pltpu.sync_copy(row_hbm.at[pl.ds(base, _CR)], ridx)
            pltpu.sync_copy(col_hbm.at[pl.ds(base, _CR)], cidx)
            pltpu.sync_copy(w_hbm.at[pl.ds(base, _CR)], wbuf)

            def gbody(g, gc):
                pltpu.sync_copy(xs.at[ridx.at[g]], rows)
                for j in range(8):
                    iv = cidx[g, pl.ds(j * 16, 16)]
                    cidx2[g, pl.ds(j * 16, 16)] = _remap_half(iv, lo)
                    wv = wbuf[g, pl.ds(j * 16, 16)]
                    for u in range(16):
                        i = j * 16 + u
                        rows[i, :] = rows[i, :] * wv[u]
                pltpu.sync_copy(rows, shared.at[cidx2.at[g]], add=True)
                return gc

            lax.fori_loop(0, _CR, gbody, 0)
            return carry

        lax.fori_loop(0, _CH, chunk, 0)
        plsc.subcore_barrier()
        pltpu.sync_copy(
            shared.at[pl.ds(s * _ZR, _ZR)],
            out_hbm.at[c, s3, pl.ds(s * _ZR, _ZR)],
        )
        plsc.subcore_barrier()


def _prep_body(degn_ref, xt_ref, xs_ref):
    deg = degn_ref[:, 0:1] + 1.0
    dinv = lax.rsqrt(deg)
    xs_ref[...] = xt_ref[...] * dinv


def _dense_body(acc_ref, xs_ref, degn_ref, wz_ref, bz_ref, wh_ref, bh_ref,
                probs_ref, wlin_ref, blin_ref, out_ref):
    deg = degn_ref[:, 0:1] + 1.0
    dinv = lax.rsqrt(deg)
    probs = probs_ref[...]
    h = jnp.zeros((_B, _HID), jnp.float32)
    for s3 in range(3):
        ys = (acc_ref[s3] + xs_ref[:, s3 * 16:(s3 + 1) * 16]) * dinv
        for q in range(4):
            p = s3 * 4 + q
            yp = ys[:, q * 4:(q + 1) * 4]
            z = jnp.dot(yp, wz_ref[...], preferred_element_type=jnp.float32) \
                + bz_ref[...]
            ht = jnp.tanh(
                jnp.dot(yp, wh_ref[...], preferred_element_type=jnp.float32)
                + bh_ref[...])
            h = h + probs[0:1, p:p + 1] * (1.0 - jax.nn.sigmoid(z)) * ht
    out_ref[...] = jnp.dot(jnp.maximum(h, 0.0), wlin_ref[...],
                           preferred_element_type=jnp.float32) + blin_ref[...]


def kernel(x, edge_index, edge_attr, Wcz, bcz, Wcr, bcr, Wch, bch,
           Wlz, blz, Wlr, blr, Wlh, blh, attention, Wlin, blin):
    n = x.shape[0]

    xt = jnp.transpose(x, (0, 2, 1)).reshape(n, 48)
    pad = _EPAD - _E
    row_p = jnp.concatenate(
        [edge_index[0], jnp.zeros((pad,), jnp.int32)]).reshape(_ROWS, 128)
    col_p = jnp.concatenate(
        [edge_index[1], jnp.zeros((pad,), jnp.int32)]).reshape(_ROWS, 128)
    w_p = jnp.concatenate(
        [edge_attr, jnp.zeros((pad,), jnp.float32)]).reshape(_ROWS, 128)

    wlz_t = Wlz[:_HID]
    wz_eff = Wcz @ wlz_t
    bz_eff = (bcz @ wlz_t + blz)[None, :]
    wlh_t = Wlh[:_HID]
    wh_eff = Wch @ wlh_t
    bh_eff = (bch @ wlh_t + blh)[None, :]
    probs = jax.nn.softmax(attention)[None, :]
    blin2 = blin[None, :]

    degp = _deg_kernel(col_p, w_p)
    degn = degp.reshape(_NP, 16)[:n, :8]

    xs48 = pl.pallas_call(
        _prep_body,
        grid=(n // _B,),
        in_specs=[
            pl.BlockSpec((_B, 8), lambda i: (i, 0)),
            pl.BlockSpec((_B, 48), lambda i: (i, 0)),
        ],
        out_specs=pl.BlockSpec((_B, 48), lambda i: (i, 0)),
        out_shape=jax.ShapeDtypeStruct((n, 48), jnp.float32),
    )(degn, xt)

    xs_sl = [
        jnp.pad(xs48[:, 16 * s3:16 * (s3 + 1)], ((0, _NP - n), (0, 0)))
        for s3 in range(3)
    ]

    accp = _agg_kernel(row_p, col_p, w_p, xs_sl[0], xs_sl[1], xs_sl[2])
    acc3 = jnp.transpose(accp, (1, 0, 2, 3)).reshape(3, _NP, 16)[:, :n]

    out = pl.pallas_call(
        _dense_body,
        grid=(n // _B,),
        in_specs=[
            pl.BlockSpec((3, _B, 16), lambda i: (0, i, 0)),
            pl.BlockSpec((_B, 48), lambda i: (i, 0)),
            pl.BlockSpec((_B, 8), lambda i: (i, 0)),
            pl.BlockSpec((4, _HID), lambda i: (0, 0)),
            pl.BlockSpec((1, _HID), lambda i: (0, 0)),
            pl.BlockSpec((4, _HID), lambda i: (0, 0)),
            pl.BlockSpec((1, _HID), lambda i: (0, 0)),
            pl.BlockSpec((1, _PERIODS), lambda i: (0, 0)),
            pl.BlockSpec((_HID, _PERIODS), lambda i: (0, 0)),
            pl.BlockSpec((1, _PERIODS), lambda i: (0, 0)),
        ],
        out_specs=pl.BlockSpec((_B, _PERIODS), lambda i: (i, 0)),
        out_shape=jax.ShapeDtypeStruct((n, _PERIODS), jnp.float32),
    )(acc3, xs48, degn, wz_eff, bz_eff, wh_eff, bh_eff, probs, Wlin, blin2)
    return out


# speedup vs baseline: 60.3922x; 1.0125x over previous
import functools

import jax
import jax.numpy as jnp
from jax import lax
from jax.experimental import pallas as pl
from jax.experimental.pallas import tpu as pltpu
from jax.experimental.pallas import tpu_sc as plsc

_N = 100000
_E = 3200000
_HID = 32
_PERIODS = 12

_NSUB = 16
_NCORE = 2
_NP = 100352
_NPH = _NP // 2
_ZR = _NPH // _NSUB
_CR = 8
_RPT = 1568
_CH = _RPT // _CR
_ROWS = _NSUB * _RPT
_EPAD = _ROWS * 128
_B = 2000


def _sc_mesh():
    return plsc.VectorSubcoreMesh(core_axis_name="c", subcore_axis_name="s")


def _remap_half(idx, lo):
    idx2 = idx - lo
    ok = jnp.logical_and(idx2 >= 0, idx2 < _NPH)
    return jnp.where(ok, idx2, _NPH)


@functools.partial(
    pl.kernel,
    out_type=pltpu.HBM((_NCORE, _NPH, 16), jnp.float32),
    mesh=_sc_mesh(),
    compiler_params=pltpu.CompilerParams(use_tc_tiling_on_sc=False),
    scratch_types=[
        pltpu.VMEM((_CR, 128), jnp.int32),
        pltpu.VMEM((_CR, 128), jnp.int32),
        pltpu.VMEM((_CR, 128), jnp.int32),
        pltpu.VMEM((_CR, 128), jnp.float32),
        pltpu.VMEM((_CR * 128, 16), jnp.float32),
        pltpu.VMEM((_CR * 128, 16), jnp.float32),
        pltpu.VMEM((784, 16), jnp.float32),
        pltpu.VMEM_SHARED((_NPH + 8, 16), jnp.float32),
        pltpu.SemaphoreType.DMA,
        pltpu.SemaphoreType.DMA,
    ],
)
def _deg_kernel(col_hbm, w_hbm, out_hbm, cidx, cidx2a, cidx2b, wbuf,
                rows_a, rows_b, zbuf, shared, ssema, ssemb):
    c = lax.axis_index("c")
    s = lax.axis_index("s")
    lo = c * _NPH

    zero16 = jnp.zeros((16,), jnp.float32)

    def zb(i, carry):
        zbuf[i, :] = zero16
        return carry

    lax.fori_loop(0, 784, zb, 0)
    for i4 in range(4):
        pltpu.sync_copy(zbuf, shared.at[pl.ds((s * 4 + i4) * 784, 784)])
    plsc.subcore_barrier()

    bufs = ((rows_a, cidx2a, ssema), (rows_b, cidx2b, ssemb))

    def do_chunk(ch, par, t):
        rows, cidx2, ssem = bufs[par]

        @pl.when(t > 0)
        def _drain():
            for g in range(_CR):
                pltpu.make_async_copy(
                    rows.at[pl.ds(g * 128, 128)],
                    shared.at[cidx2.at[g]], ssem).wait()

        base = s * _RPT + ch * _CR
        pltpu.sync_copy(col_hbm.at[pl.ds(base, _CR)], cidx)
        pltpu.sync_copy(w_hbm.at[pl.ds(base, _CR)], wbuf)

        def build_g(g, carry):
            for j in range(8):
                iv = cidx[g, pl.ds(j * 16, 16)]
                cidx2[g, pl.ds(j * 16, 16)] = _remap_half(iv, lo)
                wv = wbuf[g, pl.ds(j * 16, 16)]
                for u in range(16):
                    i = g * 128 + j * 16 + u
                    rows[i, :] = jnp.full((16,), wv[u], jnp.float32)
            return carry

        lax.fori_loop(0, _CR, build_g, 0)
        for g in range(_CR):
            pltpu.async_copy(rows.at[pl.ds(g * 128, 128)],
                             shared.at[cidx2.at[g]], ssem, add=True)

    def pair(t, carry):
        do_chunk(2 * t, 0, t)
        do_chunk(2 * t + 1, 1, t)
        return carry

    lax.fori_loop(0, _CH // 2, pair, 0)
    for par in range(2):
        rows, cidx2, ssem = bufs[par]
        for g in range(_CR):
            pltpu.make_async_copy(
                rows.at[pl.ds(g * 128, 128)],
                shared.at[cidx2.at[g]], ssem).wait()
    plsc.subcore_barrier()
    pltpu.sync_copy(
        shared.at[pl.ds(s * _ZR, _ZR)], out_hbm.at[c, pl.ds(s * _ZR, _ZR)]
    )


@functools.partial(
    pl.kernel,
    out_type=pltpu.HBM((_NCORE, 3, _NPH, 16), jnp.float32),
    mesh=_sc_mesh(),
    compiler_params=pltpu.CompilerParams(use_tc_tiling_on_sc=False),
    scratch_types=[
        pltpu.VMEM((_CR, 128), jnp.int32),
        pltpu.VMEM((_CR, 128), jnp.int32),
        pltpu.VMEM((_CR, 128), jnp.int32),
        pltpu.VMEM((_CR, 128), jnp.int32),
        pltpu.VMEM((_CR, 128), jnp.float32),
        pltpu.VMEM((_CR * 128, 16), jnp.float32),
        pltpu.VMEM((_CR * 128, 16), jnp.float32),
        pltpu.VMEM((784, 16), jnp.float32),
        pltpu.VMEM_SHARED((_NPH + 8, 16), jnp.float32),
        pltpu.SemaphoreType.DMA,
        pltpu.SemaphoreType.DMA,
        pltpu.SemaphoreType.DMA,
    ],
)
def _agg_kernel(row_hbm, col_hbm, w_hbm, xs0, xs1, xs2, out_hbm,
                ridx, cidx, cidx2a, cidx2b, wbuf, rows_a, rows_b, zbuf,
                shared, gsem, ssema, ssemb):
    c = lax.axis_index("c")
    s = lax.axis_index("s")
    lo = c * _NPH

    zero16 = jnp.zeros((16,), jnp.float32)

    def zb(i, carry):
        zbuf[i, :] = zero16
        return carry

    lax.fori_loop(0, 784, zb, 0)

    bufs = ((rows_a, cidx2a, ssema), (rows_b, cidx2b, ssemb))

    def do_chunk(xs, ch, par, t):
        rows, cidx2, ssem = bufs[par]

        @pl.when(t > 0)
        def _drain():
            for g in range(_CR):
                pltpu.make_async_copy(
                    rows.at[pl.ds(g * 128, 128)],
                    shared.at[cidx2.at[g]], ssem).wait()

        base = s * _RPT + ch * _CR
        pltpu.sync_copy(row_hbm.at[pl.ds(base, _CR)], ridx)
        pltpu.sync_copy(col_hbm.at[pl.ds(base, _CR)], cidx)
        pltpu.sync_copy(w_hbm.at[pl.ds(base, _CR)], wbuf)
        for g in range(_CR):
            pltpu.async_copy(xs.at[ridx.at[g]],
                             rows.at[pl.ds(g * 128, 128)], gsem)
        for g in range(_CR):
            pltpu.make_async_copy(xs.at[ridx.at[g]],
                                  rows.at[pl.ds(g * 128, 128)], gsem).wait()

        def scale_g(g, carry):
            for j in range(8):
                iv = cidx[g, pl.ds(j * 16, 16)]
                cidx2[g, pl.ds(j * 16, 16)] = _remap_half(iv, lo)
                wv = wbuf[g, pl.ds(j * 16, 16)]
                for u in range(16):
                    i = g * 128 + j * 16 + u
                    rows[i, :] = rows[i, :] * wv[u]
            return carry

        lax.fori_loop(0, _CR, scale_g, 0)
        for g in range(_CR):
            pltpu.async_copy(rows.at[pl.ds(g * 128, 128)],
                             shared.at[cidx2.at[g]], ssem, add=True)

    for s3, xs in enumerate((xs0, xs1, xs2)):
        for i4 in range(4):
            pltpu.sync_copy(zbuf, shared.at[pl.ds((s * 4 + i4) * 784, 784)])
        plsc.subcore_barrier()

        def pair(t, carry):
            do_chunk(xs, 2 * t, 0, t)
            do_chunk(xs, 2 * t + 1, 1, t)
            return carry

        lax.fori_loop(0, _CH // 2, pair, 0)
        for par in range(2):
            rows, cidx2, ssem = bufs[par]
            for g in range(_CR):
                pltpu.make_async_copy(
                    rows.at[pl.ds(g * 128, 128)],
                    shared.at[cidx2.at[g]], ssem).wait()
        plsc.subcore_barrier()
        pltpu.sync_copy(
            shared.at[pl.ds(s * _ZR, _ZR)],
            out_hbm.at[c, s3, pl.ds(s * _ZR, _ZR)],
        )
        plsc.subcore_barrier()


def _prep_body(degn_ref, xt_ref, xs_ref):
    deg = degn_ref[:, 0:1] + 1.0
    dinv = lax.rsqrt(deg)
    xs_ref[...] = xt_ref[...] * dinv


def _dense_body(acc_ref, xs_ref, degn_ref, wz_ref, bz_ref, wh_ref, bh_ref,
                probs_ref, wlin_ref, blin_ref, out_ref):
    deg = degn_ref[:, 0:1] + 1.0
    dinv = lax.rsqrt(deg)
    probs = probs_ref[...]
    h = jnp.zeros((_B, _HID), jnp.float32)
    for s3 in range(3):
        ys = (acc_ref[s3] + xs_ref[:, s3 * 16:(s3 + 1) * 16]) * dinv
        for q in range(4):
            p = s3 * 4 + q
            yp = ys[:, q * 4:(q + 1) * 4]
            z = jnp.dot(yp, wz_ref[...], preferred_element_type=jnp.float32) \
                + bz_ref[...]
            ht = jnp.tanh(
                jnp.dot(yp, wh_ref[...], preferred_element_type=jnp.float32)
                + bh_ref[...])
            h = h + probs[0:1, p:p + 1] * (1.0 - jax.nn.sigmoid(z)) * ht
    out_ref[...] = jnp.dot(jnp.maximum(h, 0.0), wlin_ref[...],
                           preferred_element_type=jnp.float32) + blin_ref[...]


def kernel(x, edge_index, edge_attr, Wcz, bcz, Wcr, bcr, Wch, bch,
           Wlz, blz, Wlr, blr, Wlh, blh, attention, Wlin, blin):
    n = x.shape[0]

    xt = jnp.transpose(x, (0, 2, 1)).reshape(n, 48)
    pad = _EPAD - _E
    row_p = jnp.concatenate(
        [edge_index[0], jnp.zeros((pad,), jnp.int32)]).reshape(_ROWS, 128)
    col_p = jnp.concatenate(
        [edge_index[1], jnp.zeros((pad,), jnp.int32)]).reshape(_ROWS, 128)
    w_p = jnp.concatenate(
        [edge_attr, jnp.zeros((pad,), jnp.float32)]).reshape(_ROWS, 128)

    wlz_t = Wlz[:_HID]
    wz_eff = Wcz @ wlz_t
    bz_eff = (bcz @ wlz_t + blz)[None, :]
    wlh_t = Wlh[:_HID]
    wh_eff = Wch @ wlh_t
    bh_eff = (bch @ wlh_t + blh)[None, :]
    probs = jax.nn.softmax(attention)[None, :]
    blin2 = blin[None, :]

    degp = _deg_kernel(col_p, w_p)
    degn = degp.reshape(_NP, 16)[:n, :8]

    xs48 = pl.pallas_call(
        _prep_body,
        grid=(n // _B,),
        in_specs=[
            pl.BlockSpec((_B, 8), lambda i: (i, 0)),
            pl.BlockSpec((_B, 48), lambda i: (i, 0)),
        ],
        out_specs=pl.BlockSpec((_B, 48), lambda i: (i, 0)),
        out_shape=jax.ShapeDtypeStruct((n, 48), jnp.float32),
    )(degn, xt)

    xs_sl = [
        jnp.pad(xs48[:, 16 * s3:16 * (s3 + 1)], ((0, _NP - n), (0, 0)))
        for s3 in range(3)
    ]

    accp = _agg_kernel(row_p, col_p, w_p, xs_sl[0], xs_sl[1], xs_sl[2])
    acc3 = jnp.transpose(accp, (1, 0, 2, 3)).reshape(3, _NP, 16)[:, :n]

    out = pl.pallas_call(
        _dense_body,
        grid=(n // _B,),
        in_specs=[
            pl.BlockSpec((3, _B, 16), lambda i: (0, i, 0)),
            pl.BlockSpec((_B, 48), lambda i: (i, 0)),
            pl.BlockSpec((_B, 8), lambda i: (i, 0)),
            pl.BlockSpec((4, _HID), lambda i: (0, 0)),
            pl.BlockSpec((1, _HID), lambda i: (0, 0)),
            pl.BlockSpec((4, _HID), lambda i: (0, 0)),
            pl.BlockSpec((1, _HID), lambda i: (0, 0)),
            pl.BlockSpec((1, _PERIODS), lambda i: (0, 0)),
            pl.BlockSpec((_HID, _PERIODS), lambda i: (0, 0)),
            pl.BlockSpec((1, _PERIODS), lambda i: (0, 0)),
        ],
        out_specs=pl.BlockSpec((_B, _PERIODS), lambda i: (i, 0)),
        out_shape=jax.ShapeDtypeStruct((n, _PERIODS), jnp.float32),
    )(acc3, xs48, degn, wz_eff, bz_eff, wh_eff, bh_eff, probs, Wlin, blin2)
    return out
